# baseline (speedup 1.0000x reference)
import jax
import jax.numpy as jnp
from jax.experimental import pallas as pl

N_DOCS = 100000
M = 16
K = 256
SUB = 8
D = M * SUB
BATCH = 1024
NEG_TOP_K = 200

NB = 2048
GRID1 = (N_DOCS + NB - 1) // NB
P = GRID1 * NB
NEG_INF = -1e30


def _scores_body(q_ref, cent_t_ref, codes_t_ref, pos_ref,
                 s_ref, mx_ref, mn_ref, su_ref, sq_ref, rp_ref):
    j = pl.program_id(0)
    codes = codes_t_ref[...]
    rows = []
    for m in range(M):
        oh = (jax.lax.broadcasted_iota(jnp.int32, (K, NB), 0)
              == codes[m, :][None, :]).astype(jnp.float32)
        rows.append(
            jax.lax.dot(cent_t_ref[m], oh,
                        preferred_element_type=jnp.float32))
    e_t = jnp.concatenate(rows, axis=0)
    s = jax.lax.dot(q_ref[...], e_t, preferred_element_type=jnp.float32)
    col = j * NB + jax.lax.broadcasted_iota(jnp.int32, (1, NB), 1)
    real = col < N_DOCS
    s = jnp.where(real, s, NEG_INF)
    s_ref[...] = s
    mx_ref[...] = jnp.max(s, axis=1, keepdims=True)[None]
    mn_ref[...] = jnp.min(jnp.where(real, s, 1e30), axis=1, keepdims=True)[None]
    sr = jnp.where(real, s, 0.0)
    su_ref[...] = jnp.sum(sr, axis=1, keepdims=True)[None]
    sq_ref[...] = jnp.sum(sr * sr, axis=1, keepdims=True)[None]
    pos = pos_ref[0, :]
    rp_ref[...] = jnp.sum(
        jnp.where(col == pos[:, None], s, 0.0), axis=1, keepdims=True)[None]


def _compute_scores(query_embeds, centroids, codes, pos_pids):
    codes32 = codes.astype(jnp.int32)
    codes_t = jnp.pad(codes32, ((0, P - N_DOCS), (0, 0))).T
    cent_t = jnp.transpose(centroids, (0, 2, 1))
    pos2d = pos_pids.astype(jnp.int32).reshape(1, BATCH)
    part = jax.ShapeDtypeStruct((GRID1, BATCH, 1), jnp.float32)
    return pl.pallas_call(
        _scores_body,
        grid=(GRID1,),
        in_specs=[
            pl.BlockSpec((BATCH, D), lambda j: (0, 0)),
            pl.BlockSpec((M, SUB, K), lambda j: (0, 0, 0)),
            pl.BlockSpec((M, NB), lambda j: (0, j)),
            pl.BlockSpec((1, BATCH), lambda j: (0, 0)),
        ],
        out_specs=[
            pl.BlockSpec((BATCH, NB), lambda j: (0, j)),
            pl.BlockSpec((1, BATCH, 1), lambda j: (j, 0, 0)),
            pl.BlockSpec((1, BATCH, 1), lambda j: (j, 0, 0)),
            pl.BlockSpec((1, BATCH, 1), lambda j: (j, 0, 0)),
            pl.BlockSpec((1, BATCH, 1), lambda j: (j, 0, 0)),
            pl.BlockSpec((1, BATCH, 1), lambda j: (j, 0, 0)),
        ],
        out_shape=[jax.ShapeDtypeStruct((BATCH, P), jnp.float32),
                   part, part, part, part, part],
    )(query_embeds, cent_t, codes_t, pos2d)


QB = 32
NQ = BATCH // QB
SEL_ITERS = 3
NT = 8
NLAD = 8


def _loss_body(mx_ref, mn_ref, su_ref, sq_ref, rp_ref, s_ref, out_ref):
    g = pl.program_id(0)
    s = s_ref[...]
    m = jnp.max(mx_ref[...], axis=0)
    mn = jnp.min(mn_ref[...], axis=0)
    rel = jnp.sum(rp_ref[...], axis=0)
    mu = jnp.sum(su_ref[...], axis=0) / N_DOCS
    var = jnp.sum(sq_ref[...], axis=0) / N_DOCS - mu * mu
    sd = jnp.sqrt(jnp.maximum(var, 0.0))

    def refine(carry, thresholds):
        lo, hi = carry
        for t in thresholds:
            c = jnp.sum((s >= t).astype(jnp.float32), axis=1, keepdims=True)
            ge = c >= NEG_TOP_K
            lo = jnp.where(ge, jnp.maximum(lo, t), lo)
            hi = jnp.where(ge, hi, jnp.minimum(hi, t))
        return lo, hi

    ladder = [mu + sd * (2.0 + 0.4 * j) for j in range(NLAD)]
    lo, hi = refine((mn, m + 1.0), ladder)

    def body(_, carry):
        lo, hi = carry
        step = (hi - lo) / (NT + 1)
        return refine((lo, hi), [lo + step * (j + 1) for j in range(NT)])

    lo, hi = jax.lax.fori_loop(0, SEL_ITERS, body, (lo, hi))

    ex = jnp.exp(s - m)
    ge_hi = s >= hi
    c_hi = jnp.sum(ge_hi.astype(jnp.float32), axis=1, keepdims=True)
    sum_hi = jnp.sum(jnp.where(ge_hi, ex, 0.0), axis=1, keepdims=True)
    total = sum_hi + (NEG_TOP_K - c_hi) * jnp.exp(lo - m)
    row_loss = jnp.log(jnp.exp(rel - m) + total) + m - rel

    @pl.when(g == 0)
    def _():
        out_ref[...] = jnp.zeros_like(out_ref)

    out_ref[...] += (jnp.sum(row_loss) / BATCH).reshape(1, 1)


def kernel(query_embeds, centroids, codes, pos_pids):
    s, mx, mn, su, sq, rp = _compute_scores(
        query_embeds, centroids, codes, pos_pids)
    stat_spec = pl.BlockSpec((GRID1, QB, 1), lambda g: (0, g, 0))
    out = pl.pallas_call(
        _loss_body,
        grid=(NQ,),
        in_specs=[stat_spec, stat_spec, stat_spec, stat_spec, stat_spec,
                  pl.BlockSpec((QB, P), lambda g: (g, 0))],
        out_specs=pl.BlockSpec((1, 1), lambda g: (0, 0)),
        out_shape=jax.ShapeDtypeStruct((1, 1), jnp.float32),
    )(mx, mn, su, sq, rp, s)
    return out[0, 0]


# speedup vs baseline: 1.2055x; 1.2055x over previous
import jax
import jax.numpy as jnp
from jax.experimental import pallas as pl

N_DOCS = 100000
M = 16
K = 256
SUB = 8
D = M * SUB
BATCH = 1024
NEG_TOP_K = 200

NB = 2048
GRID1 = (N_DOCS + NB - 1) // NB
P = GRID1 * NB
NEG_INF = -1e30


def _scores_body(q_ref, cent_t_ref, codes_t_ref, pos_ref,
                 s_ref, mx_ref, mn_ref, su_ref, sq_ref, rp_ref):
    j = pl.program_id(0)
    codes = codes_t_ref[...]
    rows = []
    for m in range(M):
        oh = (jax.lax.broadcasted_iota(jnp.int32, (K, NB), 0)
              == codes[m, :][None, :]).astype(jnp.float32)
        rows.append(
            jax.lax.dot(cent_t_ref[m], oh,
                        preferred_element_type=jnp.float32))
    e_t = jnp.concatenate(rows, axis=0)
    s = jax.lax.dot(q_ref[...], e_t, preferred_element_type=jnp.float32)
    col = j * NB + jax.lax.broadcasted_iota(jnp.int32, (1, NB), 1)
    real = col < N_DOCS
    s = jnp.where(real, s, NEG_INF)
    s_ref[...] = s
    mx_ref[...] = jnp.max(s, axis=1, keepdims=True)[None]
    mn_ref[...] = jnp.min(jnp.where(real, s, 1e30), axis=1, keepdims=True)[None]
    sr = jnp.where(real, s, 0.0)
    su_ref[...] = jnp.sum(sr, axis=1, keepdims=True)[None]
    sq_ref[...] = jnp.sum(sr * sr, axis=1, keepdims=True)[None]
    pos = pos_ref[0, :]
    rp_ref[...] = jnp.sum(
        jnp.where(col == pos[:, None], s, 0.0), axis=1, keepdims=True)[None]


def _compute_scores(query_embeds, centroids, codes, pos_pids):
    codes32 = codes.astype(jnp.int32)
    codes_t = jnp.pad(codes32, ((0, P - N_DOCS), (0, 0))).T
    cent_t = jnp.transpose(centroids, (0, 2, 1))
    pos2d = pos_pids.astype(jnp.int32).reshape(1, BATCH)
    part = jax.ShapeDtypeStruct((GRID1, BATCH, 1), jnp.float32)
    return pl.pallas_call(
        _scores_body,
        grid=(GRID1,),
        in_specs=[
            pl.BlockSpec((BATCH, D), lambda j: (0, 0)),
            pl.BlockSpec((M, SUB, K), lambda j: (0, 0, 0)),
            pl.BlockSpec((M, NB), lambda j: (0, j)),
            pl.BlockSpec((1, BATCH), lambda j: (0, 0)),
        ],
        out_specs=[
            pl.BlockSpec((BATCH, NB), lambda j: (0, j)),
            pl.BlockSpec((1, BATCH, 1), lambda j: (j, 0, 0)),
            pl.BlockSpec((1, BATCH, 1), lambda j: (j, 0, 0)),
            pl.BlockSpec((1, BATCH, 1), lambda j: (j, 0, 0)),
            pl.BlockSpec((1, BATCH, 1), lambda j: (j, 0, 0)),
            pl.BlockSpec((1, BATCH, 1), lambda j: (j, 0, 0)),
        ],
        out_shape=[jax.ShapeDtypeStruct((BATCH, P), jnp.float32),
                   part, part, part, part, part],
    )(query_embeds, cent_t, codes_t, pos2d)


QB = 32
NQ = BATCH // QB
SEL_ITERS = 4
NT = 4
NLAD = 8


def _loss_body(mx_ref, mn_ref, su_ref, sq_ref, rp_ref, s_ref, out_ref):
    g = pl.program_id(0)
    s = s_ref[...]
    m = jnp.max(mx_ref[...], axis=0)
    mn = jnp.min(mn_ref[...], axis=0)
    rel = jnp.sum(rp_ref[...], axis=0)
    mu = jnp.sum(su_ref[...], axis=0) / N_DOCS
    var = jnp.sum(sq_ref[...], axis=0) / N_DOCS - mu * mu
    sd = jnp.sqrt(jnp.maximum(var, 0.0))

    def refine(carry, thresholds):
        lo, hi = carry
        for t in thresholds:
            c = jnp.sum((s >= t).astype(jnp.float32), axis=1, keepdims=True)
            ge = c >= NEG_TOP_K
            lo = jnp.where(ge, jnp.maximum(lo, t), lo)
            hi = jnp.where(ge, hi, jnp.minimum(hi, t))
        return lo, hi

    ladder = [mu + sd * (2.0 + 0.4 * j) for j in range(NLAD)]
    lo, hi = refine((mn, m + 1.0), ladder)

    def body(_, carry):
        lo, hi = carry
        step = (hi - lo) / (NT + 1)
        return refine((lo, hi), [lo + step * (j + 1) for j in range(NT)])

    lo, hi = jax.lax.fori_loop(0, SEL_ITERS, body, (lo, hi))

    ex = jnp.exp(s - m)
    ge_hi = s >= hi
    c_hi = jnp.sum(ge_hi.astype(jnp.float32), axis=1, keepdims=True)
    sum_hi = jnp.sum(jnp.where(ge_hi, ex, 0.0), axis=1, keepdims=True)
    total = sum_hi + (NEG_TOP_K - c_hi) * jnp.exp(0.5 * (lo + hi) - m)
    row_loss = jnp.log(jnp.exp(rel - m) + total) + m - rel

    @pl.when(g == 0)
    def _():
        out_ref[...] = jnp.zeros_like(out_ref)

    out_ref[...] += (jnp.sum(row_loss) / BATCH).reshape(1, 1)


def kernel(query_embeds, centroids, codes, pos_pids):
    s, mx, mn, su, sq, rp = _compute_scores(
        query_embeds, centroids, codes, pos_pids)
    stat_spec = pl.BlockSpec((GRID1, QB, 1), lambda g: (0, g, 0))
    out = pl.pallas_call(
        _loss_body,
        grid=(NQ,),
        in_specs=[stat_spec, stat_spec, stat_spec, stat_spec, stat_spec,
                  pl.BlockSpec((QB, P), lambda g: (g, 0))],
        out_specs=pl.BlockSpec((1, 1), lambda g: (0, 0)),
        out_shape=jax.ShapeDtypeStruct((1, 1), jnp.float32),
    )(mx, mn, su, sq, rp, s)
    return out[0, 0]
